# baseline (speedup 1.0000x reference)
import functools

import jax
import jax.numpy as jnp
from jax import lax
from jax.experimental import pallas as pl
from jax.experimental.pallas import tpu as pltpu
from jax.experimental.pallas import tpu_sc as plsc

_EPS = 1e-12
_NC = 2
_NS = 16
_NW = _NC * _NS


def _sc_gather(idx_flat, table, chunk=128):
    n = idx_flat.shape[0]
    h = table.shape[1]
    per_w = n // _NW
    n_chunks = per_w // chunk
    assert n_chunks * chunk == per_w and n_chunks >= 2
    n2 = n_chunks // 2
    odd = n_chunks % 2 == 1
    mesh = plsc.VectorSubcoreMesh(core_axis_name="c", subcore_axis_name="s")

    @functools.partial(
        pl.kernel,
        mesh=mesh,
        out_type=jax.ShapeDtypeStruct((n, h), table.dtype),
        scratch_types=[
            pltpu.VMEM((per_w,), jnp.int32),
            pltpu.VMEM((chunk, h), table.dtype),
            pltpu.VMEM((chunk, h), table.dtype),
            pltpu.SemaphoreType.DMA,
            pltpu.SemaphoreType.DMA,
            pltpu.SemaphoreType.DMA,
            pltpu.SemaphoreType.DMA,
        ],
    )
    def gather_kernel(idx_hbm, table_hbm, out_hbm, idx_v, r0, r1,
                      sg0, sg1, so0, so1):
        wid = lax.axis_index("s") * _NC + lax.axis_index("c")
        base = wid * per_w
        pltpu.sync_copy(idx_hbm.at[pl.ds(base, per_w)], idx_v)

        def gather_start(i, buf, sem):
            pltpu.make_async_copy(
                table_hbm.at[idx_v.at[pl.ds(i * chunk, chunk)]], buf, sem
            ).start()

        def gather_wait(i, buf, sem):
            pltpu.make_async_copy(
                table_hbm.at[idx_v.at[pl.ds(i * chunk, chunk)]], buf, sem
            ).wait()

        def out_start(i, buf, sem):
            pltpu.make_async_copy(
                buf, out_hbm.at[pl.ds(base + i * chunk, chunk)], sem
            ).start()

        def out_wait(buf, sem):
            pltpu.make_async_copy(
                buf, out_hbm.at[pl.ds(base, chunk)], sem
            ).wait()

        gather_start(0, r0, sg0)

        @pl.loop(0, n2)
        def _(k):
            i0 = 2 * k

            @pl.when(k > 0)
            def _():
                out_wait(r1, so1)

            gather_start(i0 + 1, r1, sg1)
            gather_wait(i0, r0, sg0)
            out_start(i0, r0, so0)
            out_wait(r0, so0)

            @pl.when(k < n2 - 1)
            def _():
                gather_start(i0 + 2, r0, sg0)

            gather_wait(i0 + 1, r1, sg1)
            out_start(i0 + 1, r1, so1)

        out_wait(r1, so1)
        if odd:
            i_last = n_chunks - 1
            gather_start(i_last, r0, sg0)
            gather_wait(i_last, r0, sg0)
            out_start(i_last, r0, so0)
            out_wait(r0, so0)

    return gather_kernel(idx_flat, table)


def _tc_body(*refs):
    tt_ref, gath_ref, pos_ref, comb_ref, gamma_ref, beta_ref = refs[:6]
    out_ref = refs[-1]
    bb, l, h = gath_ref.shape
    nt = comb_ref.shape[0]
    tt = tt_ref[...]
    onehot = (
        tt[:, :, None] == lax.broadcasted_iota(jnp.int32, (1, 1, nt), 2)
    ).astype(jnp.float32)
    extra = lax.dot_general(
        onehot.reshape(bb * l, nt),
        comb_ref[...],
        dimension_numbers=(((1,), (0,)), ((), ())),
        preferred_element_type=jnp.float32,
    )
    emb = (gath_ref[...] + pos_ref[...][None, :, :]).reshape(bb * l, h) + extra
    ones_h = jnp.ones((h, h), jnp.float32)
    dn = (((1,), (0,)), ((), ()))
    mu = lax.dot_general(
        emb, ones_h, dimension_numbers=dn,
        preferred_element_type=jnp.float32) * (1.0 / h)
    ex2 = lax.dot_general(
        emb * emb, ones_h, dimension_numbers=dn,
        preferred_element_type=jnp.float32) * (1.0 / h)
    var = ex2 - mu * mu
    norm = (emb - mu) * lax.rsqrt(var + _EPS)
    out = norm * gamma_ref[...] + beta_ref[...]
    out_ref[...] = out.reshape(bb, l, h)


_TC_PARAMS = pltpu.CompilerParams(dimension_semantics=("parallel",))


def _tc_finish_slice(tt_s, gathered_s, pos, comb, gamma, beta, big, s_blk,
                     out_full_shape, bb=16, interpret=False):
    bs, l = tt_s.shape
    h = pos.shape[-1]
    nt = comb.shape[0]
    nblk = bs // bb
    grid = (nblk,)
    in_specs = [
        pl.BlockSpec((bb, l), lambda i: (i, 0)),
        pl.BlockSpec((bb, l, h), lambda i: (i, 0, 0)),
        pl.BlockSpec((l, h), lambda i: (0, 0)),
        pl.BlockSpec((nt, h), lambda i: (0, 0)),
        pl.BlockSpec((1, h), lambda i: (0, 0)),
        pl.BlockSpec((1, h), lambda i: (0, 0)),
    ]
    args = [tt_s, gathered_s, pos, comb, gamma, beta]
    io_aliases = {}
    if big is not None:
        args.append(big)
        in_specs.append(pl.BlockSpec(memory_space=pl.ANY))
        io_aliases = {6: 0}
    return pl.pallas_call(
        _tc_body,
        grid=grid,
        in_specs=in_specs,
        out_specs=pl.BlockSpec(
            (bb, l, h), lambda i, s_blk=s_blk: (s_blk + i, 0, 0)),
        out_shape=jax.ShapeDtypeStruct(out_full_shape, jnp.float32),
        input_output_aliases=io_aliases,
        compiler_params=None if interpret else _TC_PARAMS,
        interpret=interpret,
    )(*args)


def kernel(input_ids, token_type_ids, word_embeddings, position_embeddings,
           token_type_embeddings, sentence_type_embeddings, gamma, beta):
    b, l = input_ids.shape
    h = word_embeddings.shape[1]
    ids_flat = input_ids.astype(jnp.int32).reshape(b * l)
    tt = token_type_ids.astype(jnp.int32)

    ns = sentence_type_embeddings.shape[0]
    tok_rows = jnp.take(
        token_type_embeddings,
        (jnp.arange(ns) > 0).astype(jnp.int32), axis=0)
    comb = sentence_type_embeddings + tok_rows
    comb = jnp.concatenate(
        [comb, jnp.zeros((32 - ns, h), jnp.float32)], axis=0)

    pos = position_embeddings[:l]
    gamma2 = gamma.reshape(1, h)
    beta2 = beta.reshape(1, h)

    n_slices = 8
    bb = 64
    bs = b // n_slices
    big = None
    for s in range(n_slices):
        gathered_s = _sc_gather(
            ids_flat[s * bs * l:(s + 1) * bs * l], word_embeddings
        ).reshape(bs, l, h)
        big = _tc_finish_slice(
            tt[s * bs:(s + 1) * bs], gathered_s, pos, comb, gamma2, beta2,
            big, s * (bs // bb), (b, l, h), bb=bb)
    return big


# speedup vs baseline: 1.0182x; 1.0182x over previous
import functools

import jax
import jax.numpy as jnp
from jax import lax
from jax.experimental import pallas as pl
from jax.experimental.pallas import tpu as pltpu
from jax.experimental.pallas import tpu_sc as plsc

_EPS = 1e-12
_NC = 2
_NS = 16
_NW = _NC * _NS


def _sc_gather(idx_flat, table, chunk=128):
    n = idx_flat.shape[0]
    h = table.shape[1]
    per_w = n // _NW
    n_chunks = per_w // chunk
    assert n_chunks * chunk == per_w and n_chunks >= 2
    n2 = n_chunks // 2
    odd = n_chunks % 2 == 1
    mesh = plsc.VectorSubcoreMesh(core_axis_name="c", subcore_axis_name="s")

    @functools.partial(
        pl.kernel,
        mesh=mesh,
        out_type=jax.ShapeDtypeStruct((n, h), table.dtype),
        scratch_types=[
            pltpu.VMEM((per_w,), jnp.int32),
            pltpu.VMEM((chunk, h), table.dtype),
            pltpu.VMEM((chunk, h), table.dtype),
            pltpu.SemaphoreType.DMA,
            pltpu.SemaphoreType.DMA,
            pltpu.SemaphoreType.DMA,
            pltpu.SemaphoreType.DMA,
        ],
    )
    def gather_kernel(idx_hbm, table_hbm, out_hbm, idx_v, r0, r1,
                      sg0, sg1, so0, so1):
        wid = lax.axis_index("s") * _NC + lax.axis_index("c")
        base = wid * per_w
        pltpu.sync_copy(idx_hbm.at[pl.ds(base, per_w)], idx_v)

        def gather_start(i, buf, sem):
            pltpu.make_async_copy(
                table_hbm.at[idx_v.at[pl.ds(i * chunk, chunk)]], buf, sem
            ).start()

        def gather_wait(i, buf, sem):
            pltpu.make_async_copy(
                table_hbm.at[idx_v.at[pl.ds(i * chunk, chunk)]], buf, sem
            ).wait()

        def out_start(i, buf, sem):
            pltpu.make_async_copy(
                buf, out_hbm.at[pl.ds(base + i * chunk, chunk)], sem
            ).start()

        def out_wait(buf, sem):
            pltpu.make_async_copy(
                buf, out_hbm.at[pl.ds(base, chunk)], sem
            ).wait()

        gather_start(0, r0, sg0)

        @pl.loop(0, n2)
        def _(k):
            i0 = 2 * k

            @pl.when(k > 0)
            def _():
                out_wait(r1, so1)

            gather_start(i0 + 1, r1, sg1)
            gather_wait(i0, r0, sg0)
            out_start(i0, r0, so0)
            out_wait(r0, so0)

            @pl.when(k < n2 - 1)
            def _():
                gather_start(i0 + 2, r0, sg0)

            gather_wait(i0 + 1, r1, sg1)
            out_start(i0 + 1, r1, so1)

        out_wait(r1, so1)
        if odd:
            i_last = n_chunks - 1
            gather_start(i_last, r0, sg0)
            gather_wait(i_last, r0, sg0)
            out_start(i_last, r0, so0)
            out_wait(r0, so0)

    return gather_kernel(idx_flat, table)


def _tc_body(*refs):
    tt_ref, gath_ref, pos_ref, comb_ref, gamma_ref, beta_ref = refs[:6]
    out_ref = refs[-1]
    bb, l, h = gath_ref.shape
    nt = comb_ref.shape[0]
    tt = tt_ref[...]
    onehot = (
        tt[:, :, None] == lax.broadcasted_iota(jnp.int32, (1, 1, nt), 2)
    ).astype(jnp.float32)
    extra = lax.dot_general(
        onehot.reshape(bb * l, nt),
        comb_ref[...],
        dimension_numbers=(((1,), (0,)), ((), ())),
        preferred_element_type=jnp.float32,
    )
    emb = (gath_ref[...] + pos_ref[...][None, :, :]).reshape(bb * l, h) + extra
    ones_h = jnp.ones((h, h), jnp.float32)
    dn = (((1,), (0,)), ((), ()))
    mu = lax.dot_general(
        emb, ones_h, dimension_numbers=dn,
        preferred_element_type=jnp.float32) * (1.0 / h)
    ex2 = lax.dot_general(
        emb * emb, ones_h, dimension_numbers=dn,
        preferred_element_type=jnp.float32) * (1.0 / h)
    var = ex2 - mu * mu
    norm = (emb - mu) * lax.rsqrt(var + _EPS)
    out = norm * gamma_ref[...] + beta_ref[...]
    out_ref[...] = out.reshape(bb, l, h)


_TC_PARAMS = pltpu.CompilerParams(dimension_semantics=("parallel",))


def _tc_finish_slice(tt_s, gathered_s, pos, comb, gamma, beta, big, s_blk,
                     out_full_shape, bb=16, interpret=False):
    bs, l = tt_s.shape
    h = pos.shape[-1]
    nt = comb.shape[0]
    nblk = bs // bb
    grid = (nblk,)
    in_specs = [
        pl.BlockSpec((bb, l), lambda i: (i, 0)),
        pl.BlockSpec((bb, l, h), lambda i: (i, 0, 0)),
        pl.BlockSpec((l, h), lambda i: (0, 0)),
        pl.BlockSpec((nt, h), lambda i: (0, 0)),
        pl.BlockSpec((1, h), lambda i: (0, 0)),
        pl.BlockSpec((1, h), lambda i: (0, 0)),
    ]
    args = [tt_s, gathered_s, pos, comb, gamma, beta]
    io_aliases = {}
    if big is not None:
        args.append(big)
        in_specs.append(pl.BlockSpec(memory_space=pl.ANY))
        io_aliases = {6: 0}
    return pl.pallas_call(
        _tc_body,
        grid=grid,
        in_specs=in_specs,
        out_specs=pl.BlockSpec(
            (bb, l, h), lambda i, s_blk=s_blk: (s_blk + i, 0, 0)),
        out_shape=jax.ShapeDtypeStruct(out_full_shape, jnp.float32),
        input_output_aliases=io_aliases,
        compiler_params=None if interpret else _TC_PARAMS,
        interpret=interpret,
    )(*args)


def kernel(input_ids, token_type_ids, word_embeddings, position_embeddings,
           token_type_embeddings, sentence_type_embeddings, gamma, beta):
    b, l = input_ids.shape
    h = word_embeddings.shape[1]
    ids_flat = input_ids.astype(jnp.int32).reshape(b * l)
    tt = token_type_ids.astype(jnp.int32)

    ns = sentence_type_embeddings.shape[0]
    tok_rows = jnp.take(
        token_type_embeddings,
        (jnp.arange(ns) > 0).astype(jnp.int32), axis=0)
    comb = sentence_type_embeddings + tok_rows
    comb = jnp.concatenate(
        [comb, jnp.zeros((32 - ns, h), jnp.float32)], axis=0)

    pos = position_embeddings[:l]
    gamma2 = gamma.reshape(1, h)
    beta2 = beta.reshape(1, h)

    n_slices = 4
    bb = 64
    bs = b // n_slices
    big = None
    for s in range(n_slices):
        gathered_s = _sc_gather(
            ids_flat[s * bs * l:(s + 1) * bs * l], word_embeddings
        ).reshape(bs, l, h)
        big = _tc_finish_slice(
            tt[s * bs:(s + 1) * bs], gathered_s, pos, comb, gamma2, beta2,
            big, s * (bs // bb), (b, l, h), bb=bb)
    return big
